# baseline (speedup 1.0000x reference)
import functools

import jax
import jax.numpy as jnp
from jax.experimental import pallas as pl
from jax.experimental.pallas import tpu as pltpu

MEM = 8192
BS = 2048
SC = 512
TC_ = 256
VPAD = 128


def _tables_body(h_ref, Wk_ref, bk_ref, Wv_ref, bv_ref, kn_ref, v_ref):
    hbar = jnp.mean(h_ref[...], axis=0).astype(jnp.bfloat16)
    k = jnp.dot(hbar, Wk_ref[...],
                preferred_element_type=jnp.float32) + bk_ref[0]
    n = jnp.sqrt(jnp.sum(k * k, axis=-1, keepdims=True))
    kn_ref[...] = (k / jnp.maximum(n, 1e-12)).astype(jnp.bfloat16)
    v = jnp.dot(hbar, Wv_ref[...],
                preferred_element_type=jnp.float32) + bv_ref[0]
    rows = v.shape[0]
    vcols = v.shape[-1]
    pad = jnp.concatenate(
        [v, jnp.ones((rows, 1), jnp.float32),
         jnp.zeros((rows, VPAD - vcols - 1), jnp.float32)], axis=-1)
    v_ref[...] = pad.astype(jnp.bfloat16)


def _attn_body(h_ref, Wq_ref, bq_ref, kn_ref, v_ref,
               Wor_ref, Woh_ref, bo_ref, out_ref, *, S, V):
    h = h_ref[0].astype(jnp.bfloat16)
    q = jnp.dot(h, Wq_ref[...],
                preferred_element_type=jnp.float32) + bq_ref[0]
    n = jnp.sqrt(jnp.sum(q * q, axis=-1, keepdims=True))
    qn = (q / jnp.maximum(n, 1e-12)).astype(jnp.bfloat16)
    rfull = jnp.zeros((h.shape[0], VPAD), jnp.float32)
    for c in range(S // SC):
        rows = pl.ds(c * SC, SC)
        sim = jax.lax.dot_general(qn, kn_ref[rows, :],
                                  (((1,), (1,)), ((), ())),
                                  preferred_element_type=jnp.float32)
        e = jnp.exp(sim).astype(jnp.bfloat16)
        rfull += jnp.dot(e, v_ref[rows, :],
                         preferred_element_type=jnp.float32)
    denom = rfull[:, V:V + 1]
    out = jnp.dot(rfull.astype(jnp.bfloat16), Wor_ref[...],
                  preferred_element_type=jnp.float32) / denom
    out += jnp.dot(h, Woh_ref[...], preferred_element_type=jnp.float32)
    out_ref[0] = out + bo_ref[0]


@jax.jit
def kernel(hidden_states, Wq, bq, Wk, bk, Wv, bv, Wo, bo):
    B, S, H = hidden_states.shape
    K = Wq.shape[1]
    V = Wv.shape[1]
    assert MEM % S == 0 and B > 1 and S % BS == 0 and S % SC == 0
    nblk = S // BS

    bf = jnp.bfloat16
    Wor = jnp.zeros((VPAD, H), jnp.float32).at[:V].set(Wo[:V]).astype(bf)
    Woh = Wo[V:].astype(bf)
    Wq_bf = Wq.astype(bf)
    Wk_bf = Wk.astype(bf)
    Wv_bf = Wv.astype(bf)

    tconst = lambda i: (0, 0)
    kn, vals = pl.pallas_call(
        _tables_body,
        grid=(S // TC_,),
        in_specs=[
            pl.BlockSpec((B, TC_, H), lambda i: (0, i, 0)),
            pl.BlockSpec((H, K), tconst),
            pl.BlockSpec((1, K), tconst),
            pl.BlockSpec((H, V), tconst),
            pl.BlockSpec((1, V), tconst),
        ],
        out_specs=[pl.BlockSpec((TC_, K), lambda i: (i, 0)),
                   pl.BlockSpec((TC_, VPAD), lambda i: (i, 0))],
        out_shape=[jax.ShapeDtypeStruct((S, K), bf),
                   jax.ShapeDtypeStruct((S, VPAD), bf)],
    )(hidden_states, Wk_bf, bk.reshape(1, K), Wv_bf, bv.reshape(1, V))

    const = lambda b, i: (0, 0)
    out = pl.pallas_call(
        functools.partial(_attn_body, S=S, V=V),
        grid=(B, nblk),
        in_specs=[
            pl.BlockSpec((1, BS, H), lambda b, i: (b, i, 0)),
            pl.BlockSpec((H, K), const),
            pl.BlockSpec((1, K), const),
            pl.BlockSpec((S, K), const),
            pl.BlockSpec((S, VPAD), const),
            pl.BlockSpec((VPAD, H), const),
            pl.BlockSpec((H, H), const),
            pl.BlockSpec((1, H), const),
        ],
        out_specs=pl.BlockSpec((1, BS, H), lambda b, i: (b, i, 0)),
        out_shape=jax.ShapeDtypeStruct((B, S, H), jnp.float32),
    )(hidden_states, Wq_bf, bq.reshape(1, K), kn, vals, Wor, Woh,
      bo.reshape(1, H))
    return out


# speedup vs baseline: 1.1000x; 1.1000x over previous
import functools

import jax
import jax.numpy as jnp
from jax.experimental import pallas as pl
from jax.experimental.pallas import tpu as pltpu

MEM = 8192
BS = 2048
SC = 512
VPAD = 64


def _tables_body(h_ref, Wk_ref, bk_ref, Wv_ref, bv_ref, kn_ref, v_ref):
    hbar = jnp.mean(h_ref[...], axis=0).astype(jnp.bfloat16)
    k = jnp.dot(hbar, Wk_ref[...],
                preferred_element_type=jnp.float32) + bk_ref[0]
    n = jnp.sqrt(jnp.sum(k * k, axis=-1, keepdims=True))
    kn_ref[...] = (k / jnp.maximum(n, 1e-12)).astype(jnp.bfloat16)
    v = jnp.dot(hbar, Wv_ref[...],
                preferred_element_type=jnp.float32) + bv_ref[0]
    v_ref[...] = v.astype(jnp.bfloat16)


def _attn_body(h_ref, Wq_ref, bq_ref, kn_ref, v_ref,
               Wor_ref, Woh_ref, bo_ref, out_ref, *, S, V):
    h = h_ref[0].astype(jnp.bfloat16)
    q = jnp.dot(h, Wq_ref[...],
                preferred_element_type=jnp.float32) + bq_ref[0]
    n = jnp.sqrt(jnp.sum(q * q, axis=-1, keepdims=True))
    qn = (q / jnp.maximum(n, 1e-12)).astype(jnp.bfloat16)
    rfull = jnp.zeros((h.shape[0], VPAD), jnp.float32)
    denom = jnp.zeros((h.shape[0], 1), jnp.float32)
    for c in range(S // SC):
        rows = pl.ds(c * SC, SC)
        sim = jax.lax.dot_general(qn, kn_ref[rows, :],
                                  (((1,), (1,)), ((), ())),
                                  preferred_element_type=jnp.float32)
        ef = jnp.exp(sim)
        e = ef.astype(jnp.bfloat16)
        denom += jnp.sum(ef, axis=-1, keepdims=True)
        rfull += jnp.dot(e, v_ref[rows, :],
                         preferred_element_type=jnp.float32)
    out = jnp.dot(rfull.astype(jnp.bfloat16), Wor_ref[...],
                  preferred_element_type=jnp.float32) / denom
    out += jnp.dot(h, Woh_ref[...], preferred_element_type=jnp.float32)
    out_ref[0] = out + bo_ref[0]


@jax.jit
def kernel(hidden_states, Wq, bq, Wk, bk, Wv, bv, Wo, bo):
    B, S, H = hidden_states.shape
    K = Wq.shape[1]
    V = Wv.shape[1]
    assert MEM % S == 0 and B > 1 and S % BS == 0 and S % SC == 0
    nblk = S // BS

    bf = jnp.bfloat16
    Wor = Wo[:V].astype(bf)
    Woh = Wo[V:].astype(bf)
    Wq_bf = Wq.astype(bf)
    Wk_bf = Wk.astype(bf)
    Wv_bf = Wv.astype(bf)

    kn, vals = pl.pallas_call(
        _tables_body,
        out_shape=[jax.ShapeDtypeStruct((S, K), bf),
                   jax.ShapeDtypeStruct((S, VPAD), bf)],
    )(hidden_states, Wk_bf, bk.reshape(1, K), Wv_bf, bv.reshape(1, V))

    const = lambda b, i: (0, 0)
    out = pl.pallas_call(
        functools.partial(_attn_body, S=S, V=V),
        grid=(B, nblk),
        in_specs=[
            pl.BlockSpec((1, BS, H), lambda b, i: (b, i, 0)),
            pl.BlockSpec((H, K), const),
            pl.BlockSpec((1, K), const),
            pl.BlockSpec((S, K), const),
            pl.BlockSpec((S, VPAD), const),
            pl.BlockSpec((VPAD, H), const),
            pl.BlockSpec((H, H), const),
            pl.BlockSpec((1, H), const),
        ],
        out_specs=pl.BlockSpec((1, BS, H), lambda b, i: (b, i, 0)),
        out_shape=jax.ShapeDtypeStruct((B, S, H), jnp.float32),
    )(hidden_states, Wq_bf, bq.reshape(1, K), kn, vals, Wor, Woh,
      bo.reshape(1, H))
    return out


# speedup vs baseline: 1.2164x; 1.1059x over previous
import functools

import jax
import jax.numpy as jnp
from jax.experimental import pallas as pl
from jax.experimental.pallas import tpu as pltpu

MEM = 8192
SC = 512


def _body(h_ref, Wq_ref, bq_ref, Wk_ref, bk_ref, Wv_ref, bv_ref,
          Wor_ref, Woh_ref, bo_ref, out_ref, kn_s, v_s, *, S, V):
    g = pl.program_id(0)

    @pl.when(g == 0)
    def _tables():
        hbar = jnp.mean(h_ref[...], axis=0).astype(jnp.bfloat16)
        k = jnp.dot(hbar, Wk_ref[...],
                    preferred_element_type=jnp.float32) + bk_ref[0]
        n = jnp.sqrt(jnp.sum(k * k, axis=-1, keepdims=True))
        kn_s[...] = (k / jnp.maximum(n, 1e-12)).astype(jnp.bfloat16)
        v = jnp.dot(hbar, Wv_ref[...],
                    preferred_element_type=jnp.float32) + bv_ref[0]
        v_s[...] = v.astype(jnp.bfloat16)

    @pl.when(g > 0)
    def _attn():
        h = h_ref[g - 1].astype(jnp.bfloat16)
        q = jnp.dot(h, Wq_ref[...],
                    preferred_element_type=jnp.float32) + bq_ref[0]
        n = jnp.sqrt(jnp.sum(q * q, axis=-1, keepdims=True))
        qn = (q / jnp.maximum(n, 1e-12)).astype(jnp.bfloat16)
        rfull = jnp.zeros((S, V), jnp.float32)
        denom = jnp.zeros((S, 1), jnp.float32)
        for c in range(S // SC):
            rows = pl.ds(c * SC, SC)
            sim = jax.lax.dot_general(qn, kn_s[rows, :],
                                      (((1,), (1,)), ((), ())),
                                      preferred_element_type=jnp.float32)
            ef = jnp.exp(sim)
            denom += jnp.sum(ef, axis=-1, keepdims=True)
            rfull += jnp.dot(ef.astype(jnp.bfloat16), v_s[rows, :],
                             preferred_element_type=jnp.float32)
        out = jnp.dot(rfull.astype(jnp.bfloat16), Wor_ref[...],
                      preferred_element_type=jnp.float32) / denom
        out += jnp.dot(h, Woh_ref[...], preferred_element_type=jnp.float32)
        out_ref[0] = out + bo_ref[0]


@jax.jit
def kernel(hidden_states, Wq, bq, Wk, bk, Wv, bv, Wo, bo):
    B, S, H = hidden_states.shape
    K = Wq.shape[1]
    V = Wv.shape[1]
    assert MEM % S == 0 and B > 1 and S % SC == 0

    bf = jnp.bfloat16
    Wor = Wo[:V].astype(bf)
    Woh = Wo[V:].astype(bf)
    Wq_bf = Wq.astype(bf)
    Wk_bf = Wk.astype(bf)
    Wv_bf = Wv.astype(bf)

    const = lambda g: (0, 0)
    out = pl.pallas_call(
        functools.partial(_body, S=S, V=V),
        grid=(1 + B,),
        in_specs=[
            pl.BlockSpec((B, S, H), lambda g: (0, 0, 0)),
            pl.BlockSpec((H, K), const),
            pl.BlockSpec((1, K), const),
            pl.BlockSpec((H, K), const),
            pl.BlockSpec((1, K), const),
            pl.BlockSpec((H, V), const),
            pl.BlockSpec((1, V), const),
            pl.BlockSpec((V, H), const),
            pl.BlockSpec((H, H), const),
            pl.BlockSpec((1, H), const),
        ],
        out_specs=pl.BlockSpec((1, S, H),
                               lambda g: (jnp.maximum(g - 1, 0), 0, 0)),
        out_shape=jax.ShapeDtypeStruct((B, S, H), jnp.float32),
        scratch_shapes=[
            pltpu.VMEM((S, K), bf),
            pltpu.VMEM((S, V), bf),
        ],
    )(hidden_states, Wq_bf, bq.reshape(1, K), Wk_bf, bk.reshape(1, K),
      Wv_bf, bv.reshape(1, V), Wor, Woh, bo.reshape(1, H))
    return out
